# Initial kernel scaffold; baseline (speedup 1.0000x reference)
import functools

import jax
import jax.numpy as jnp
from jax import lax
from jax.experimental import pallas as pl
from jax.experimental.pallas import tpu as pltpu
from jax.experimental.pallas import tpu_sc as plsc

N = 10000
E = 320000
F_IN = 128
F_HID = 128
F_OUT = 64


NC = 2
NS = 16
NW = NC * NS
C = 128
CHUNKS = E // C
BASE_CHUNKS = CHUNKS // NW
EXTRA = CHUNKS % NW
ROWS_PER_TILE = N // NS


def _make_sc_agg(F, compute_deg):
  mesh = plsc.VectorSubcoreMesh(core_axis_name="c", subcore_axis_name="s")
  out_type = [jax.ShapeDtypeStruct((N, F), jnp.float32),
              jax.ShapeDtypeStruct((N, F), jnp.float32)]
  scratch = [
      pltpu.VMEM((C,), jnp.int32),
      pltpu.VMEM((C,), jnp.int32),
      pltpu.VMEM((C, F), jnp.float32),
      pltpu.VMEM_SHARED((N, F), jnp.float32),
      pltpu.SemaphoreType.DMA,
  ]
  if compute_deg:
    out_type += [jax.ShapeDtypeStruct((N,), jnp.float32),
                 jax.ShapeDtypeStruct((N,), jnp.float32)]
    scratch += [
        pltpu.VMEM((C,), jnp.float32),
        pltpu.VMEM_SHARED((N,), jnp.float32),
    ]

  def body(y_hbm, src_hbm, dst_hbm, z2d_hbm, z1_hbm,
           agg0_hbm, agg1_hbm, *rest):
    if compute_deg:
      (deg0_hbm, deg1_hbm, src_v, dst_v, rows_v, agg_sh, sem,
       ones_v, deg_sh) = rest
    else:
      src_v, dst_v, rows_v, agg_sh, sem = rest
    c = lax.axis_index("c")
    s = lax.axis_index("s")
    wid = s * NC + c

    r0 = s * ROWS_PER_TILE
    pltpu.sync_copy(z2d_hbm.at[pl.ds(r0, ROWS_PER_TILE)],
                    agg_sh.at[pl.ds(r0, ROWS_PER_TILE)])
    if compute_deg:
      @pl.when(s == 0)
      def _():
        pltpu.sync_copy(z1_hbm, deg_sh)
      ones16 = jnp.full((16,), 1.0, jnp.float32)
      for j in range(C // 16):
        ones_v[pl.ds(j * 16, 16)] = ones16
    plsc.subcore_barrier()

    def do_chunk(g):
      off = pl.multiple_of(g * C, 8)
      pltpu.sync_copy(src_hbm.at[pl.ds(off, C)], src_v)
      pltpu.sync_copy(dst_hbm.at[pl.ds(off, C)], dst_v)
      pltpu.async_copy(y_hbm.at[src_v], rows_v, sem).wait()
      pltpu.sync_copy(rows_v, agg_sh.at[dst_v], add=True)
      if compute_deg:
        pltpu.sync_copy(ones_v, deg_sh.at[dst_v], add=True)

    def loop_body(j, carry):
      do_chunk(j * NW + wid)
      return carry
    lax.fori_loop(0, BASE_CHUNKS, loop_body, 0)
    if EXTRA:
      @pl.when(wid < EXTRA)
      def _():
        do_chunk(BASE_CHUNKS * NW + wid)

    plsc.subcore_barrier()

    @pl.when(c == 0)
    def _():
      pltpu.sync_copy(agg_sh.at[pl.ds(r0, ROWS_PER_TILE)],
                      agg0_hbm.at[pl.ds(r0, ROWS_PER_TILE)])
    @pl.when(c == 1)
    def _():
      pltpu.sync_copy(agg_sh.at[pl.ds(r0, ROWS_PER_TILE)],
                      agg1_hbm.at[pl.ds(r0, ROWS_PER_TILE)])
    if compute_deg:
      @pl.when((c == 0) & (s == 0))
      def _():
        pltpu.sync_copy(deg_sh, deg0_hbm)
      @pl.when((c == 1) & (s == 0))
      def _():
        pltpu.sync_copy(deg_sh, deg1_hbm)

  return pl.kernel(body, out_type=out_type, mesh=mesh, scratch_types=scratch)


_sc_agg128_deg = _make_sc_agg(F_HID, compute_deg=True)
_sc_agg128 = _make_sc_agg(F_HID, compute_deg=False)
_sc_agg64 = _make_sc_agg(F_OUT, compute_deg=False)


BM = 512
GRID = pl.cdiv(N, BM)


def _tc_first_body(x_ref, ws_ref, wn_ref, b_ref, s_ref, y_ref):
  xb = x_ref[...]
  s_ref[...] = jnp.dot(xb, ws_ref[...],
                       preferred_element_type=jnp.float32) + b_ref[...]
  y_ref[...] = jnp.dot(xb, wn_ref[...], preferred_element_type=jnp.float32)


def _tc_first(x, ws, wn, b):
  fo = ws.shape[1]
  return pl.pallas_call(
      _tc_first_body,
      grid=(GRID,),
      in_specs=[
          pl.BlockSpec((BM, F_IN), lambda i: (i, 0)),
          pl.BlockSpec((F_IN, fo), lambda i: (0, 0)),
          pl.BlockSpec((F_IN, fo), lambda i: (0, 0)),
          pl.BlockSpec((1, fo), lambda i: (0, 0)),
      ],
      out_specs=[
          pl.BlockSpec((BM, fo), lambda i: (i, 0)),
          pl.BlockSpec((BM, fo), lambda i: (i, 0)),
      ],
      out_shape=[
          jax.ShapeDtypeStruct((N, fo), jnp.float32),
          jax.ShapeDtypeStruct((N, fo), jnp.float32),
      ],
  )(x, ws, wn, b)


def _tc_mid_body(sp_ref, a0_ref, a1_ref, d0_ref, d1_ref, ws_ref, wn_ref,
                 b_ref, s_ref, y_ref):
  deg = jnp.maximum(d0_ref[...] + d1_ref[...], 1.0)
  agg = a0_ref[...] + a1_ref[...]
  h = jax.nn.relu(sp_ref[...] + agg / deg[:, None])
  s_ref[...] = jnp.dot(h, ws_ref[...],
                       preferred_element_type=jnp.float32) + b_ref[...]
  y_ref[...] = jnp.dot(h, wn_ref[...], preferred_element_type=jnp.float32)


def _tc_mid(sp, a0, a1, d0, d1, ws, wn, b):
  fi = ws.shape[0]
  fo = ws.shape[1]
  return pl.pallas_call(
      _tc_mid_body,
      grid=(GRID,),
      in_specs=[
          pl.BlockSpec((BM, fi), lambda i: (i, 0)),
          pl.BlockSpec((BM, fi), lambda i: (i, 0)),
          pl.BlockSpec((BM, fi), lambda i: (i, 0)),
          pl.BlockSpec((BM,), lambda i: (i,)),
          pl.BlockSpec((BM,), lambda i: (i,)),
          pl.BlockSpec((fi, fo), lambda i: (0, 0)),
          pl.BlockSpec((fi, fo), lambda i: (0, 0)),
          pl.BlockSpec((1, fo), lambda i: (0, 0)),
      ],
      out_specs=[
          pl.BlockSpec((BM, fo), lambda i: (i, 0)),
          pl.BlockSpec((BM, fo), lambda i: (i, 0)),
      ],
      out_shape=[
          jax.ShapeDtypeStruct((N, fo), jnp.float32),
          jax.ShapeDtypeStruct((N, fo), jnp.float32),
      ],
  )(sp, a0, a1, d0, d1, ws, wn, b)


def _tc_last_body(sp_ref, a0_ref, a1_ref, d0_ref, d1_ref, o_ref):
  deg = jnp.maximum(d0_ref[...] + d1_ref[...], 1.0)
  agg = a0_ref[...] + a1_ref[...]
  o_ref[...] = sp_ref[...] + agg / deg[:, None]


def _tc_last(sp, a0, a1, d0, d1):
  fo = sp.shape[1]
  return pl.pallas_call(
      _tc_last_body,
      grid=(GRID,),
      in_specs=[
          pl.BlockSpec((BM, fo), lambda i: (i, 0)),
          pl.BlockSpec((BM, fo), lambda i: (i, 0)),
          pl.BlockSpec((BM, fo), lambda i: (i, 0)),
          pl.BlockSpec((BM,), lambda i: (i,)),
          pl.BlockSpec((BM,), lambda i: (i,)),
      ],
      out_specs=pl.BlockSpec((BM, fo), lambda i: (i, 0)),
      out_shape=jax.ShapeDtypeStruct((N, fo), jnp.float32),
  )(sp, a0, a1, d0, d1)


def kernel(x, edge_index, W_self0, W_neigh0, b0, W_self1, W_neigh1, b1,
           W_self2, W_neigh2, b2):
  src = edge_index[0]
  dst = edge_index[1]
  z2d128 = jnp.zeros((N, F_HID), jnp.float32)
  z2d64 = jnp.zeros((N, F_OUT), jnp.float32)
  z1 = jnp.zeros((N,), jnp.float32)

  s0, y0 = _tc_first(x, W_self0, W_neigh0, b0.reshape(1, -1))
  a0, a1, d0, d1 = _sc_agg128_deg(y0, src, dst, z2d128, z1)
  s1, y1 = _tc_mid(s0, a0, a1, d0, d1, W_self1, W_neigh1, b1.reshape(1, -1))
  a0b, a1b = _sc_agg128(y1, src, dst, z2d128, z1)
  s2, y2 = _tc_mid(s1, a0b, a1b, d0, d1, W_self2, W_neigh2, b2.reshape(1, -1))
  a0c, a1c = _sc_agg64(y2, src, dst, z2d64, z1)
  return _tc_last(s2, a0c, a1c, d0, d1)


# speedup vs baseline: 6.2885x; 6.2885x over previous
import functools

import jax
import jax.numpy as jnp
from jax import lax
from jax.experimental import pallas as pl
from jax.experimental.pallas import tpu as pltpu
from jax.experimental.pallas import tpu_sc as plsc

N = 10000
E = 320000
F_IN = 128
F_HID = 128
F_OUT = 64


NC = 2
NS = 16
NW = NC * NS
C = 128
CHUNKS = E // C
BASE_CHUNKS = CHUNKS // NW
EXTRA = CHUNKS % NW
R_MAIN = 632
R_LAST = N - (NS - 1) * R_MAIN


@functools.lru_cache(maxsize=None)
def _make_sc_agg(F, compute_deg):
  mesh = plsc.VectorSubcoreMesh(core_axis_name="c", subcore_axis_name="s",
                                num_cores=NC, num_subcores=NS)
  out_type = [jax.ShapeDtypeStruct((N, F), jnp.float32),
              jax.ShapeDtypeStruct((N, F), jnp.float32)]
  scratch = [
      pltpu.VMEM((C,), jnp.int32),
      pltpu.VMEM((C,), jnp.int32),
      pltpu.VMEM((C, F), jnp.float32),
      pltpu.VMEM_SHARED((N, F), jnp.float32),
      pltpu.SemaphoreType.DMA,
  ]
  if compute_deg:
    out_type += [jax.ShapeDtypeStruct((N,), jnp.float32),
                 jax.ShapeDtypeStruct((N,), jnp.float32)]
    scratch += [
        pltpu.VMEM((C,), jnp.float32),
        pltpu.VMEM_SHARED((N,), jnp.float32),
    ]

  def body(y_hbm, src_hbm, dst_hbm, z2d_hbm, z1_hbm,
           agg0_hbm, agg1_hbm, *rest):
    if compute_deg:
      (deg0_hbm, deg1_hbm, src_v, dst_v, rows_v, agg_sh, sem,
       ones_v, deg_sh) = rest
    else:
      src_v, dst_v, rows_v, agg_sh, sem = rest
    c = lax.axis_index("c")
    s = lax.axis_index("s")
    wid = s * NC + c

    r0 = pl.multiple_of(s * R_MAIN, 8)

    def copy_rows(src_ref, dst_ref):
      @pl.when(s < NS - 1)
      def _():
        pltpu.sync_copy(src_ref.at[pl.ds(r0, R_MAIN)],
                        dst_ref.at[pl.ds(r0, R_MAIN)])
      @pl.when(s == NS - 1)
      def _():
        pltpu.sync_copy(src_ref.at[pl.ds(r0, R_LAST)],
                        dst_ref.at[pl.ds(r0, R_LAST)])

    copy_rows(z2d_hbm, agg_sh)
    if compute_deg:
      @pl.when(s == 0)
      def _():
        pltpu.sync_copy(z1_hbm, deg_sh)
      ones16 = jnp.full((16,), 1.0, jnp.float32)
      for j in range(C // 16):
        ones_v[pl.ds(j * 16, 16)] = ones16
    plsc.subcore_barrier()

    def do_chunk(g):
      off = pl.multiple_of(g * C, 8)
      pltpu.sync_copy(src_hbm.at[pl.ds(off, C)], src_v)
      pltpu.sync_copy(dst_hbm.at[pl.ds(off, C)], dst_v)
      pltpu.async_copy(y_hbm.at[src_v], rows_v, sem).wait()
      pltpu.sync_copy(rows_v, agg_sh.at[dst_v], add=True)
      if compute_deg:
        pltpu.sync_copy(ones_v, deg_sh.at[dst_v], add=True)

    def loop_body(j, carry):
      do_chunk(j * NW + wid)
      return carry
    lax.fori_loop(0, BASE_CHUNKS, loop_body, 0)
    if EXTRA:
      @pl.when(wid < EXTRA)
      def _():
        do_chunk(BASE_CHUNKS * NW + wid)

    plsc.subcore_barrier()

    @pl.when(c == 0)
    def _():
      copy_rows(agg_sh, agg0_hbm)
    @pl.when(c == 1)
    def _():
      copy_rows(agg_sh, agg1_hbm)
    if compute_deg:
      @pl.when((c == 0) & (s == 0))
      def _():
        pltpu.sync_copy(deg_sh, deg0_hbm)
      @pl.when((c == 1) & (s == 0))
      def _():
        pltpu.sync_copy(deg_sh, deg1_hbm)

  return pl.kernel(body, out_type=out_type, mesh=mesh, scratch_types=scratch)


BM = 512
GRID = pl.cdiv(N, BM)


def _tc_first_body(x_ref, ws_ref, wn_ref, b_ref, s_ref, y_ref):
  xb = x_ref[...]
  s_ref[...] = jnp.dot(xb, ws_ref[...],
                       preferred_element_type=jnp.float32) + b_ref[...]
  y_ref[...] = jnp.dot(xb, wn_ref[...], preferred_element_type=jnp.float32)


def _tc_first(x, ws, wn, b):
  fo = ws.shape[1]
  return pl.pallas_call(
      _tc_first_body,
      grid=(GRID,),
      in_specs=[
          pl.BlockSpec((BM, F_IN), lambda i: (i, 0)),
          pl.BlockSpec((F_IN, fo), lambda i: (0, 0)),
          pl.BlockSpec((F_IN, fo), lambda i: (0, 0)),
          pl.BlockSpec((1, fo), lambda i: (0, 0)),
      ],
      out_specs=[
          pl.BlockSpec((BM, fo), lambda i: (i, 0)),
          pl.BlockSpec((BM, fo), lambda i: (i, 0)),
      ],
      out_shape=[
          jax.ShapeDtypeStruct((N, fo), jnp.float32),
          jax.ShapeDtypeStruct((N, fo), jnp.float32),
      ],
  )(x, ws, wn, b)


def _tc_mid_body(sp_ref, a0_ref, a1_ref, d0_ref, d1_ref, ws_ref, wn_ref,
                 b_ref, s_ref, y_ref):
  deg = jnp.maximum(d0_ref[...] + d1_ref[...], 1.0)
  agg = a0_ref[...] + a1_ref[...]
  h = jax.nn.relu(sp_ref[...] + agg / deg[:, None])
  s_ref[...] = jnp.dot(h, ws_ref[...],
                       preferred_element_type=jnp.float32) + b_ref[...]
  y_ref[...] = jnp.dot(h, wn_ref[...], preferred_element_type=jnp.float32)


def _tc_mid(sp, a0, a1, d0, d1, ws, wn, b):
  fi = ws.shape[0]
  fo = ws.shape[1]
  return pl.pallas_call(
      _tc_mid_body,
      grid=(GRID,),
      in_specs=[
          pl.BlockSpec((BM, fi), lambda i: (i, 0)),
          pl.BlockSpec((BM, fi), lambda i: (i, 0)),
          pl.BlockSpec((BM, fi), lambda i: (i, 0)),
          pl.BlockSpec((BM,), lambda i: (i,)),
          pl.BlockSpec((BM,), lambda i: (i,)),
          pl.BlockSpec((fi, fo), lambda i: (0, 0)),
          pl.BlockSpec((fi, fo), lambda i: (0, 0)),
          pl.BlockSpec((1, fo), lambda i: (0, 0)),
      ],
      out_specs=[
          pl.BlockSpec((BM, fo), lambda i: (i, 0)),
          pl.BlockSpec((BM, fo), lambda i: (i, 0)),
      ],
      out_shape=[
          jax.ShapeDtypeStruct((N, fo), jnp.float32),
          jax.ShapeDtypeStruct((N, fo), jnp.float32),
      ],
  )(sp, a0, a1, d0, d1, ws, wn, b)


def _tc_last_body(sp_ref, a0_ref, a1_ref, d0_ref, d1_ref, o_ref):
  deg = jnp.maximum(d0_ref[...] + d1_ref[...], 1.0)
  agg = a0_ref[...] + a1_ref[...]
  o_ref[...] = sp_ref[...] + agg / deg[:, None]


def _tc_last(sp, a0, a1, d0, d1):
  fo = sp.shape[1]
  return pl.pallas_call(
      _tc_last_body,
      grid=(GRID,),
      in_specs=[
          pl.BlockSpec((BM, fo), lambda i: (i, 0)),
          pl.BlockSpec((BM, fo), lambda i: (i, 0)),
          pl.BlockSpec((BM, fo), lambda i: (i, 0)),
          pl.BlockSpec((BM,), lambda i: (i,)),
          pl.BlockSpec((BM,), lambda i: (i,)),
      ],
      out_specs=pl.BlockSpec((BM, fo), lambda i: (i, 0)),
      out_shape=jax.ShapeDtypeStruct((N, fo), jnp.float32),
  )(sp, a0, a1, d0, d1)


def kernel(x, edge_index, W_self0, W_neigh0, b0, W_self1, W_neigh1, b1,
           W_self2, W_neigh2, b2):
  src = edge_index[0]
  dst = edge_index[1]
  z2d128 = jnp.zeros((N, F_HID), jnp.float32)
  z1 = jnp.zeros((N,), jnp.float32)
  ws2 = jnp.concatenate([W_self2, jnp.zeros_like(W_self2)], axis=1)
  wn2 = jnp.concatenate([W_neigh2, jnp.zeros_like(W_neigh2)], axis=1)
  b2p = jnp.concatenate([b2, jnp.zeros_like(b2)])

  s0, y0 = _tc_first(x, W_self0, W_neigh0, b0.reshape(1, -1))
  a0, a1, d0, d1 = _make_sc_agg(F_HID, True)(y0, src, dst, z2d128, z1)
  s1, y1 = _tc_mid(s0, a0, a1, d0, d1, W_self1, W_neigh1, b1.reshape(1, -1))
  a0b, a1b = _make_sc_agg(F_HID, False)(y1, src, dst, z2d128, z1)
  s2, y2 = _tc_mid(s1, a0b, a1b, d0, d1, ws2, wn2, b2p.reshape(1, -1))
  a0c, a1c = _make_sc_agg(F_HID, False)(y2, src, dst, z2d128, z1)
  return _tc_last(s2, a0c, a1c, d0, d1)[:, :F_OUT]


# speedup vs baseline: 11.4329x; 1.8181x over previous
import functools

import jax
import jax.numpy as jnp
from jax import lax
from jax.experimental import pallas as pl
from jax.experimental.pallas import tpu as pltpu
from jax.experimental.pallas import tpu_sc as plsc

N = 10000
E = 320000
F_IN = 128
F_HID = 128
F_OUT = 64


NC = 2
NS = 16
NW = NC * NS
C = 128
E_PAD = -(-E // (NW * C)) * NW * C
PT = E_PAD // (NW * C)
NPAD = N + 8
R_MAIN = 632
R_LAST = N - (NS - 1) * R_MAIN


@functools.lru_cache(maxsize=None)
def _make_sc_agg(F, compute_deg):
  mesh = plsc.VectorSubcoreMesh(core_axis_name="c", subcore_axis_name="s",
                                num_cores=NC, num_subcores=NS)
  out_type = [jax.ShapeDtypeStruct((N, F), jnp.float32),
              jax.ShapeDtypeStruct((N, F), jnp.float32)]
  scratch = [
      pltpu.VMEM((2, C), jnp.int32),
      pltpu.VMEM((2, C), jnp.int32),
      pltpu.VMEM((C, F), jnp.float32),
      pltpu.VMEM((C, F), jnp.float32),
      pltpu.VMEM_SHARED((NPAD, F), jnp.float32),
      pltpu.SemaphoreType.DMA,
      pltpu.SemaphoreType.DMA,
      pltpu.SemaphoreType.DMA,
      pltpu.SemaphoreType.DMA,
  ]
  if compute_deg:
    out_type += [jax.ShapeDtypeStruct((NPAD,), jnp.float32),
                 jax.ShapeDtypeStruct((NPAD,), jnp.float32)]
    scratch += [
        pltpu.VMEM((C,), jnp.float32),
        pltpu.VMEM_SHARED((NPAD,), jnp.float32),
    ]

  def body(y_hbm, ei_hbm, z2d_hbm, z1_hbm, agg0_hbm, agg1_hbm, *rest):
    if compute_deg:
      (deg0_hbm, deg1_hbm, eiv0, eiv1, rows0, rows1, agg_sh,
       gsem0, gsem1, ssem0, ssem1, ones_v, deg_sh) = rest
    else:
      eiv0, eiv1, rows0, rows1, agg_sh, gsem0, gsem1, ssem0, ssem1 = rest
    eiv = (eiv0, eiv1)
    rows = (rows0, rows1)
    gsem = (gsem0, gsem1)
    ssem = (ssem0, ssem1)
    c = lax.axis_index("c")
    s = lax.axis_index("s")
    wid = s * NC + c

    r0 = pl.multiple_of(s * R_MAIN, 8)

    def copy_rows(src_ref, dst_ref):
      @pl.when(s < NS - 1)
      def _():
        pltpu.sync_copy(src_ref.at[pl.ds(r0, R_MAIN)],
                        dst_ref.at[pl.ds(r0, R_MAIN)])
      @pl.when(s == NS - 1)
      def _():
        pltpu.sync_copy(src_ref.at[pl.ds(r0, R_LAST)],
                        dst_ref.at[pl.ds(r0, R_LAST)])

    copy_rows(z2d_hbm, agg_sh)
    if compute_deg:
      @pl.when(s == 0)
      def _():
        pltpu.sync_copy(z1_hbm, deg_sh)
      ones16 = jnp.full((16,), 1.0, jnp.float32)
      for j in range(C // 16):
        ones_v[pl.ds(j * 16, 16)] = ones16
    plsc.subcore_barrier()

    def load_and_gather(j, b):
      off = pl.multiple_of((j * NW + wid) * C, C)
      pltpu.sync_copy(ei_hbm.at[:, pl.ds(off, C)], eiv[b])
      pltpu.async_copy(y_hbm.at[eiv[b].at[0]], rows[b], gsem[b])

    def process(j, b, prefetch):
      pltpu.make_async_copy(y_hbm.at[eiv[b].at[0]], rows[b], gsem[b]).wait()
      sc1 = pltpu.async_copy(rows[b], agg_sh.at[eiv[b].at[1]], ssem[b],
                             add=True)
      if compute_deg:
        sc2 = pltpu.async_copy(ones_v, deg_sh.at[eiv[b].at[1]], ssem[b],
                               add=True)
      sc1.wait()
      if compute_deg:
        sc2.wait()
      if prefetch:
        @pl.when(j + 2 < PT)
        def _():
          load_and_gather(j + 2, b)

    load_and_gather(0, 0)
    load_and_gather(1, 1)

    def pair(jj, carry):
      for b in range(2):
        process(jj * 2 + b, b, prefetch=True)
      return carry
    lax.fori_loop(0, PT // 2, pair, 0)
    for j in range(2 * (PT // 2), PT):
      process(j, j % 2, prefetch=False)

    plsc.subcore_barrier()

    @pl.when(c == 0)
    def _():
      copy_rows(agg_sh, agg0_hbm)
    @pl.when(c == 1)
    def _():
      copy_rows(agg_sh, agg1_hbm)
    if compute_deg:
      @pl.when((c == 0) & (s == 0))
      def _():
        pltpu.sync_copy(deg_sh, deg0_hbm)
      @pl.when((c == 1) & (s == 0))
      def _():
        pltpu.sync_copy(deg_sh, deg1_hbm)

  return pl.kernel(body, out_type=out_type, mesh=mesh, scratch_types=scratch)


BM = 512
GRID = pl.cdiv(N, BM)


def _tc_first_body(x_ref, ws_ref, wn_ref, b_ref, s_ref, y_ref):
  xb = x_ref[...]
  s_ref[...] = jnp.dot(xb, ws_ref[...],
                       preferred_element_type=jnp.float32) + b_ref[...]
  y_ref[...] = jnp.dot(xb, wn_ref[...], preferred_element_type=jnp.float32)


def _tc_first(x, ws, wn, b):
  fo = ws.shape[1]
  return pl.pallas_call(
      _tc_first_body,
      grid=(GRID,),
      in_specs=[
          pl.BlockSpec((BM, F_IN), lambda i: (i, 0)),
          pl.BlockSpec((F_IN, fo), lambda i: (0, 0)),
          pl.BlockSpec((F_IN, fo), lambda i: (0, 0)),
          pl.BlockSpec((1, fo), lambda i: (0, 0)),
      ],
      out_specs=[
          pl.BlockSpec((BM, fo), lambda i: (i, 0)),
          pl.BlockSpec((BM, fo), lambda i: (i, 0)),
      ],
      out_shape=[
          jax.ShapeDtypeStruct((N, fo), jnp.float32),
          jax.ShapeDtypeStruct((N, fo), jnp.float32),
      ],
  )(x, ws, wn, b)


def _tc_mid_body(sp_ref, a0_ref, a1_ref, d0_ref, d1_ref, ws_ref, wn_ref,
                 b_ref, s_ref, y_ref):
  deg = jnp.maximum(d0_ref[...] + d1_ref[...], 1.0)
  agg = a0_ref[...] + a1_ref[...]
  h = jax.nn.relu(sp_ref[...] + agg / deg[:, None])
  s_ref[...] = jnp.dot(h, ws_ref[...],
                       preferred_element_type=jnp.float32) + b_ref[...]
  y_ref[...] = jnp.dot(h, wn_ref[...], preferred_element_type=jnp.float32)


def _tc_mid(sp, a0, a1, d0, d1, ws, wn, b):
  fi = ws.shape[0]
  fo = ws.shape[1]
  return pl.pallas_call(
      _tc_mid_body,
      grid=(GRID,),
      in_specs=[
          pl.BlockSpec((BM, fi), lambda i: (i, 0)),
          pl.BlockSpec((BM, fi), lambda i: (i, 0)),
          pl.BlockSpec((BM, fi), lambda i: (i, 0)),
          pl.BlockSpec((BM,), lambda i: (i,)),
          pl.BlockSpec((BM,), lambda i: (i,)),
          pl.BlockSpec((fi, fo), lambda i: (0, 0)),
          pl.BlockSpec((fi, fo), lambda i: (0, 0)),
          pl.BlockSpec((1, fo), lambda i: (0, 0)),
      ],
      out_specs=[
          pl.BlockSpec((BM, fo), lambda i: (i, 0)),
          pl.BlockSpec((BM, fo), lambda i: (i, 0)),
      ],
      out_shape=[
          jax.ShapeDtypeStruct((N, fo), jnp.float32),
          jax.ShapeDtypeStruct((N, fo), jnp.float32),
      ],
  )(sp, a0, a1, d0, d1, ws, wn, b)


def _tc_last_body(sp_ref, a0_ref, a1_ref, d0_ref, d1_ref, o_ref):
  deg = jnp.maximum(d0_ref[...] + d1_ref[...], 1.0)
  agg = a0_ref[...] + a1_ref[...]
  o_ref[...] = sp_ref[...] + agg / deg[:, None]


def _tc_last(sp, a0, a1, d0, d1):
  fo = sp.shape[1]
  return pl.pallas_call(
      _tc_last_body,
      grid=(GRID,),
      in_specs=[
          pl.BlockSpec((BM, fo), lambda i: (i, 0)),
          pl.BlockSpec((BM, fo), lambda i: (i, 0)),
          pl.BlockSpec((BM, fo), lambda i: (i, 0)),
          pl.BlockSpec((BM,), lambda i: (i,)),
          pl.BlockSpec((BM,), lambda i: (i,)),
      ],
      out_specs=pl.BlockSpec((BM, fo), lambda i: (i, 0)),
      out_shape=jax.ShapeDtypeStruct((N, fo), jnp.float32),
  )(sp, a0, a1, d0, d1)


def kernel(x, edge_index, W_self0, W_neigh0, b0, W_self1, W_neigh1, b1,
           W_self2, W_neigh2, b2):
  pad = E_PAD - E
  ar = jnp.arange(pad, dtype=jnp.int32)
  pad_pair = jnp.stack([(ar * 97) % N, N + (ar % 8)])
  ei = jnp.concatenate([edge_index, pad_pair], axis=1)
  z2d128 = jnp.zeros((N, F_HID), jnp.float32)
  z1 = jnp.zeros((NPAD,), jnp.float32)
  ws2 = jnp.concatenate([W_self2, jnp.zeros_like(W_self2)], axis=1)
  wn2 = jnp.concatenate([W_neigh2, jnp.zeros_like(W_neigh2)], axis=1)
  b2p = jnp.concatenate([b2, jnp.zeros_like(b2)])

  s0, y0 = _tc_first(x, W_self0, W_neigh0, b0.reshape(1, -1))
  a0, a1, d0, d1 = _make_sc_agg(F_HID, True)(y0, ei, z2d128, z1)
  d0 = d0[:N]
  d1 = d1[:N]
  s1, y1 = _tc_mid(s0, a0, a1, d0, d1, W_self1, W_neigh1, b1.reshape(1, -1))
  a0b, a1b = _make_sc_agg(F_HID, False)(y1, ei, z2d128, z1)
  s2, y2 = _tc_mid(s1, a0b, a1b, d0, d1, ws2, wn2, b2p.reshape(1, -1))
  a0c, a1c = _make_sc_agg(F_HID, False)(y2, ei, z2d128, z1)
  return _tc_last(s2, a0c, a1c, d0, d1)[:, :F_OUT]


# speedup vs baseline: 13.4063x; 1.1726x over previous
import functools

import jax
import jax.numpy as jnp
from jax import lax
from jax.experimental import pallas as pl
from jax.experimental.pallas import tpu as pltpu
from jax.experimental.pallas import tpu_sc as plsc

N = 10000
E = 320000
F_IN = 128
F_HID = 128
F_OUT = 64


NC = 2
NS = 16
NW = NC * NS
C = 128
E_PAD = -(-E // (NW * C)) * NW * C
PT = E_PAD // (NW * C)
NPAD = N + 8
NBUF = 3
DIST = NBUF - 1
R_MAIN = 632
R_LAST = N - (NS - 1) * R_MAIN


@functools.lru_cache(maxsize=None)
def _make_sc_agg(F, compute_deg):
  mesh = plsc.VectorSubcoreMesh(core_axis_name="c", subcore_axis_name="s",
                                num_cores=NC, num_subcores=NS)
  out_type = [jax.ShapeDtypeStruct((N, F), jnp.float32),
              jax.ShapeDtypeStruct((N, F), jnp.float32)]
  scratch = (
      [pltpu.VMEM((2, C), jnp.int32) for _ in range(NBUF)]
      + [pltpu.VMEM((C, F), jnp.float32) for _ in range(NBUF)]
      + [pltpu.VMEM_SHARED((NPAD, F), jnp.float32)]
      + [pltpu.SemaphoreType.DMA for _ in range(2 * NBUF)]
  )
  if compute_deg:
    out_type += [jax.ShapeDtypeStruct((NPAD,), jnp.float32),
                 jax.ShapeDtypeStruct((NPAD,), jnp.float32)]
    scratch += [
        pltpu.VMEM((C,), jnp.float32),
        pltpu.VMEM_SHARED((NPAD,), jnp.float32),
    ]

  def body(y_hbm, ei_hbm, z2d_hbm, z1_hbm, agg0_hbm, agg1_hbm, *rest):
    if compute_deg:
      deg0_hbm, deg1_hbm = rest[0], rest[1]
      rest = rest[2:]
    eiv = rest[:NBUF]
    rows = rest[NBUF:2 * NBUF]
    agg_sh = rest[2 * NBUF]
    gsem = rest[2 * NBUF + 1:3 * NBUF + 1]
    ssem = rest[3 * NBUF + 1:4 * NBUF + 1]
    if compute_deg:
      ones_v, deg_sh = rest[4 * NBUF + 1], rest[4 * NBUF + 2]
    c = lax.axis_index("c")
    s = lax.axis_index("s")
    wid = s * NC + c

    r0 = pl.multiple_of(s * R_MAIN, 8)

    def copy_rows(src_ref, dst_ref):
      @pl.when(s < NS - 1)
      def _():
        pltpu.sync_copy(src_ref.at[pl.ds(r0, R_MAIN)],
                        dst_ref.at[pl.ds(r0, R_MAIN)])
      @pl.when(s == NS - 1)
      def _():
        pltpu.sync_copy(src_ref.at[pl.ds(r0, R_LAST)],
                        dst_ref.at[pl.ds(r0, R_LAST)])

    copy_rows(z2d_hbm, agg_sh)
    if compute_deg:
      @pl.when(s == 0)
      def _():
        pltpu.sync_copy(z1_hbm, deg_sh)
      ones16 = jnp.full((16,), 1.0, jnp.float32)
      for j in range(C // 16):
        ones_v[pl.ds(j * 16, 16)] = ones16
    plsc.subcore_barrier()

    def load_and_gather(j, b):
      off = pl.multiple_of((j * NW + wid) * C, C)
      pltpu.sync_copy(ei_hbm.at[:, pl.ds(off, C)], eiv[b])
      pltpu.async_copy(y_hbm.at[eiv[b].at[0]], rows[b], gsem[b])

    def drain_scatter(b):
      pltpu.make_async_copy(rows[b], agg_sh.at[eiv[b].at[1]], ssem[b]).wait()
      if compute_deg:
        pltpu.make_async_copy(ones_v, deg_sh.at[eiv[b].at[1]], ssem[b]).wait()

    def iter_body(j, b):
      bp = (b + DIST) % NBUF
      pltpu.make_async_copy(y_hbm.at[eiv[b].at[0]], rows[b], gsem[b]).wait()
      pltpu.async_copy(rows[b], agg_sh.at[eiv[b].at[1]], ssem[b], add=True)
      if compute_deg:
        pltpu.async_copy(ones_v, deg_sh.at[eiv[b].at[1]], ssem[b], add=True)
      @pl.when(j + DIST < PT)
      def _():
        @pl.when(j >= 1)
        def _():
          drain_scatter(bp)
        load_and_gather(j + DIST, bp)

    for b in range(DIST):
      load_and_gather(b, b)

    def quad(q, carry):
      for k in range(NBUF):
        iter_body(q * NBUF + k, k)
      return carry
    lax.fori_loop(0, PT // NBUF, quad, 0)
    for j in range(NBUF * (PT // NBUF), PT):
      iter_body(j, j % NBUF)
    for j in range(max(0, PT - NBUF), PT):
      drain_scatter(j % NBUF)

    plsc.subcore_barrier()

    @pl.when(c == 0)
    def _():
      copy_rows(agg_sh, agg0_hbm)
    @pl.when(c == 1)
    def _():
      copy_rows(agg_sh, agg1_hbm)
    if compute_deg:
      @pl.when((c == 0) & (s == 0))
      def _():
        pltpu.sync_copy(deg_sh, deg0_hbm)
      @pl.when((c == 1) & (s == 0))
      def _():
        pltpu.sync_copy(deg_sh, deg1_hbm)

  return pl.kernel(body, out_type=out_type, mesh=mesh, scratch_types=scratch)


BM = 1024
GRID = pl.cdiv(N, BM)


def _tc_first_body(x_ref, ws_ref, wn_ref, b_ref, s_ref, y_ref):
  xb = x_ref[...]
  s_ref[...] = jnp.dot(xb, ws_ref[...],
                       preferred_element_type=jnp.float32) + b_ref[...]
  y_ref[...] = jnp.dot(xb, wn_ref[...], preferred_element_type=jnp.float32)


def _tc_first(x, ws, wn, b):
  fo = ws.shape[1]
  return pl.pallas_call(
      _tc_first_body,
      grid=(GRID,),
      in_specs=[
          pl.BlockSpec((BM, F_IN), lambda i: (i, 0)),
          pl.BlockSpec((F_IN, fo), lambda i: (0, 0)),
          pl.BlockSpec((F_IN, fo), lambda i: (0, 0)),
          pl.BlockSpec((1, fo), lambda i: (0, 0)),
      ],
      out_specs=[
          pl.BlockSpec((BM, fo), lambda i: (i, 0)),
          pl.BlockSpec((BM, fo), lambda i: (i, 0)),
      ],
      out_shape=[
          jax.ShapeDtypeStruct((N, fo), jnp.float32),
          jax.ShapeDtypeStruct((N, fo), jnp.float32),
      ],
  )(x, ws, wn, b)


def _tc_mid_body(sp_ref, a0_ref, a1_ref, d0_ref, d1_ref, ws_ref, wn_ref,
                 b_ref, s_ref, y_ref):
  deg = jnp.maximum(d0_ref[...] + d1_ref[...], 1.0)
  agg = a0_ref[...] + a1_ref[...]
  h = jax.nn.relu(sp_ref[...] + agg / deg[:, None])
  s_ref[...] = jnp.dot(h, ws_ref[...],
                       preferred_element_type=jnp.float32) + b_ref[...]
  y_ref[...] = jnp.dot(h, wn_ref[...], preferred_element_type=jnp.float32)


def _tc_mid(sp, a0, a1, d0, d1, ws, wn, b):
  fi = ws.shape[0]
  fo = ws.shape[1]
  return pl.pallas_call(
      _tc_mid_body,
      grid=(GRID,),
      in_specs=[
          pl.BlockSpec((BM, fi), lambda i: (i, 0)),
          pl.BlockSpec((BM, fi), lambda i: (i, 0)),
          pl.BlockSpec((BM, fi), lambda i: (i, 0)),
          pl.BlockSpec((BM,), lambda i: (i,)),
          pl.BlockSpec((BM,), lambda i: (i,)),
          pl.BlockSpec((fi, fo), lambda i: (0, 0)),
          pl.BlockSpec((fi, fo), lambda i: (0, 0)),
          pl.BlockSpec((1, fo), lambda i: (0, 0)),
      ],
      out_specs=[
          pl.BlockSpec((BM, fo), lambda i: (i, 0)),
          pl.BlockSpec((BM, fo), lambda i: (i, 0)),
      ],
      out_shape=[
          jax.ShapeDtypeStruct((N, fo), jnp.float32),
          jax.ShapeDtypeStruct((N, fo), jnp.float32),
      ],
  )(sp, a0, a1, d0, d1, ws, wn, b)


def _tc_last_body(sp_ref, a0_ref, a1_ref, d0_ref, d1_ref, o_ref):
  deg = jnp.maximum(d0_ref[...] + d1_ref[...], 1.0)
  agg = a0_ref[...] + a1_ref[...]
  o_ref[...] = sp_ref[...] + agg / deg[:, None]


def _tc_last(sp, a0, a1, d0, d1):
  fo = sp.shape[1]
  return pl.pallas_call(
      _tc_last_body,
      grid=(GRID,),
      in_specs=[
          pl.BlockSpec((BM, fo), lambda i: (i, 0)),
          pl.BlockSpec((BM, fo), lambda i: (i, 0)),
          pl.BlockSpec((BM, fo), lambda i: (i, 0)),
          pl.BlockSpec((BM,), lambda i: (i,)),
          pl.BlockSpec((BM,), lambda i: (i,)),
      ],
      out_specs=pl.BlockSpec((BM, fo), lambda i: (i, 0)),
      out_shape=jax.ShapeDtypeStruct((N, fo), jnp.float32),
  )(sp, a0, a1, d0, d1)


def kernel(x, edge_index, W_self0, W_neigh0, b0, W_self1, W_neigh1, b1,
           W_self2, W_neigh2, b2):
  pad = E_PAD - E
  ar = jnp.arange(pad, dtype=jnp.int32)
  pad_pair = jnp.stack([(ar * 97) % N, N + (ar % 8)])
  ei = jnp.concatenate([edge_index, pad_pair], axis=1)
  z2d128 = jnp.zeros((N, F_HID), jnp.float32)
  z1 = jnp.zeros((NPAD,), jnp.float32)
  ws2 = jnp.concatenate([W_self2, jnp.zeros_like(W_self2)], axis=1)
  wn2 = jnp.concatenate([W_neigh2, jnp.zeros_like(W_neigh2)], axis=1)
  b2p = jnp.concatenate([b2, jnp.zeros_like(b2)])

  s0, y0 = _tc_first(x, W_self0, W_neigh0, b0.reshape(1, -1))
  a0, a1, d0, d1 = _make_sc_agg(F_HID, True)(y0, ei, z2d128, z1)
  d0 = d0[:N]
  d1 = d1[:N]
  s1, y1 = _tc_mid(s0, a0, a1, d0, d1, W_self1, W_neigh1, b1.reshape(1, -1))
  a0b, a1b = _make_sc_agg(F_HID, False)(y1, ei, z2d128, z1)
  s2, y2 = _tc_mid(s1, a0b, a1b, d0, d1, ws2, wn2, b2p.reshape(1, -1))
  a0c, a1c = _make_sc_agg(F_HID, False)(y2, ei, z2d128, z1)
  return _tc_last(s2, a0c, a1c, d0, d1)[:, :F_OUT]


# speedup vs baseline: 14.4220x; 1.0758x over previous
import functools

import jax
import jax.numpy as jnp
from jax import lax
from jax.experimental import pallas as pl
from jax.experimental.pallas import tpu as pltpu
from jax.experimental.pallas import tpu_sc as plsc

N = 10000
E = 320000
F_IN = 128
F_HID = 128
F_OUT = 64


NC = 2
NS = 16
NW = NC * NS
C = 128
E_PAD = -(-E // (NW * C)) * NW * C
PT = E_PAD // (NW * C)
NPAD = N + 8
NBUF = 3
DIST = NBUF - 1
UNROLL = 12
R_MAIN = 632
R_LAST = N - (NS - 1) * R_MAIN


@functools.lru_cache(maxsize=None)
def _make_sc_agg(F, compute_deg):
  mesh = plsc.VectorSubcoreMesh(core_axis_name="c", subcore_axis_name="s",
                                num_cores=NC, num_subcores=NS)
  out_type = [jax.ShapeDtypeStruct((N, F), jnp.float32),
              jax.ShapeDtypeStruct((N, F), jnp.float32)]
  nib = 4 if compute_deg else 6
  scratch = (
      [pltpu.VMEM((2, C), jnp.int32) for _ in range(nib)]
      + [pltpu.VMEM((C, F), jnp.float32) for _ in range(NBUF)]
      + [pltpu.VMEM_SHARED((NPAD, F), jnp.float32)]
      + [pltpu.SemaphoreType.DMA for _ in range(nib + 2 * NBUF + 1)]
  )
  if compute_deg:
    out_type += [jax.ShapeDtypeStruct((NPAD,), jnp.float32),
                 jax.ShapeDtypeStruct((NPAD,), jnp.float32)]
    scratch += [
        pltpu.VMEM((C,), jnp.float32),
        pltpu.VMEM_SHARED((NPAD,), jnp.float32),
    ]

  def body(y_hbm, ei_hbm, z2d_hbm, z1_hbm, agg0_hbm, agg1_hbm, *rest):
    if compute_deg:
      deg0_hbm, deg1_hbm = rest[0], rest[1]
      rest = rest[2:]
    eiv = rest[:nib]
    rows = rest[nib:nib + NBUF]
    agg_sh = rest[nib + NBUF]
    sems = rest[nib + NBUF + 1:nib + NBUF + 1 + nib + 2 * NBUF + 1]
    isem = sems[:nib]
    gsem = sems[nib:nib + NBUF]
    ssem = sems[nib + NBUF:nib + 2 * NBUF]
    zsem = sems[nib + 2 * NBUF]
    if compute_deg:
      ones_v, deg_sh = rest[-2], rest[-1]
    c = lax.axis_index("c")
    s = lax.axis_index("s")
    wid = s * NC + c

    r0 = pl.multiple_of(s * R_MAIN, 8)

    def copy_rows(src_ref, dst_ref, sem=None):
      @pl.when(s < NS - 1)
      def _():
        sl = (pl.ds(r0, R_MAIN),)
        if sem is None:
          pltpu.sync_copy(src_ref.at[sl], dst_ref.at[sl])
        else:
          pltpu.async_copy(src_ref.at[sl], dst_ref.at[sl], sem)
      @pl.when(s == NS - 1)
      def _():
        sl = (pl.ds(r0, R_LAST),)
        if sem is None:
          pltpu.sync_copy(src_ref.at[sl], dst_ref.at[sl])
        else:
          pltpu.async_copy(src_ref.at[sl], dst_ref.at[sl], sem)

    def wait_rows(src_ref, dst_ref, sem):
      @pl.when(s < NS - 1)
      def _():
        sl = (pl.ds(r0, R_MAIN),)
        pltpu.make_async_copy(src_ref.at[sl], dst_ref.at[sl], sem).wait()
      @pl.when(s == NS - 1)
      def _():
        sl = (pl.ds(r0, R_LAST),)
        pltpu.make_async_copy(src_ref.at[sl], dst_ref.at[sl], sem).wait()

    def idx_load(j, ib):
      off = pl.multiple_of((j * NW + wid) * C, C)
      pltpu.async_copy(ei_hbm.at[:, pl.ds(off, C)], eiv[ib], isem[ib])

    def idx_wait(j, ib):
      off = pl.multiple_of((j * NW + wid) * C, C)
      pltpu.make_async_copy(ei_hbm.at[:, pl.ds(off, C)], eiv[ib],
                            isem[ib]).wait()

    def drain_scatter(b, ib):
      pltpu.make_async_copy(rows[b], agg_sh.at[eiv[ib].at[1]], ssem[b]).wait()
      if compute_deg:
        pltpu.make_async_copy(ones_v, deg_sh.at[eiv[ib].at[1]], ssem[b]).wait()

    copy_rows(z2d_hbm, agg_sh, zsem)
    for k in range(DIST + 1):
      idx_load(k, k)
    for k in range(DIST):
      idx_wait(k, k)
      pltpu.async_copy(y_hbm.at[eiv[k].at[0]], rows[k], gsem[k])
    wait_rows(z2d_hbm, agg_sh, zsem)
    if compute_deg:
      @pl.when(s == 0)
      def _():
        pltpu.sync_copy(z1_hbm, deg_sh)
      ones16 = jnp.full((16,), 1.0, jnp.float32)
      for j in range(C // 16):
        ones_v[pl.ds(j * 16, 16)] = ones16
    plsc.subcore_barrier()

    def iter_body(j, k):
      b = k % NBUF
      ib = k % nib
      pltpu.make_async_copy(y_hbm.at[eiv[ib].at[0]], rows[b], gsem[b]).wait()
      pltpu.async_copy(rows[b], agg_sh.at[eiv[ib].at[1]], ssem[b], add=True)
      if compute_deg:
        pltpu.async_copy(ones_v, deg_sh.at[eiv[ib].at[1]], ssem[b], add=True)
      @pl.when(j + DIST < PT)
      def _():
        @pl.when(j >= 1)
        def _():
          drain_scatter((k + NBUF - 1) % NBUF, (k + nib - 1) % nib)
        @pl.when(j + DIST + 1 < PT)
        def _():
          idx_load(j + DIST + 1, (k + DIST + 1) % nib)
        idx_wait(j + DIST, (k + DIST) % nib)
        pltpu.async_copy(y_hbm.at[eiv[(k + DIST) % nib].at[0]],
                         rows[(k + DIST) % NBUF], gsem[(k + DIST) % NBUF])

    def block(q, carry):
      for k in range(UNROLL):
        iter_body(q * UNROLL + k, k)
      return carry
    lax.fori_loop(0, PT // UNROLL, block, 0)
    for j in range(UNROLL * (PT // UNROLL), PT):
      iter_body(j, j % UNROLL)
    for j in range(max(0, PT - NBUF), PT):
      drain_scatter(j % NBUF, j % nib)

    plsc.subcore_barrier()

    @pl.when(c == 0)
    def _():
      copy_rows(agg_sh, agg0_hbm)
    @pl.when(c == 1)
    def _():
      copy_rows(agg_sh, agg1_hbm)
    if compute_deg:
      @pl.when((c == 0) & (s == 0))
      def _():
        pltpu.sync_copy(deg_sh, deg0_hbm)
      @pl.when((c == 1) & (s == 0))
      def _():
        pltpu.sync_copy(deg_sh, deg1_hbm)

  return pl.kernel(body, out_type=out_type, mesh=mesh, scratch_types=scratch)


BM = 1024
GRID = pl.cdiv(N, BM)


def _tc_first_body(x_ref, ws_ref, wn_ref, b_ref, s_ref, y_ref):
  xb = x_ref[...]
  s_ref[...] = jnp.dot(xb, ws_ref[...],
                       preferred_element_type=jnp.float32) + b_ref[...]
  y_ref[...] = jnp.dot(xb, wn_ref[...], preferred_element_type=jnp.float32)


def _tc_first(x, ws, wn, b):
  fo = ws.shape[1]
  return pl.pallas_call(
      _tc_first_body,
      grid=(GRID,),
      in_specs=[
          pl.BlockSpec((BM, F_IN), lambda i: (i, 0)),
          pl.BlockSpec((F_IN, fo), lambda i: (0, 0)),
          pl.BlockSpec((F_IN, fo), lambda i: (0, 0)),
          pl.BlockSpec((1, fo), lambda i: (0, 0)),
      ],
      out_specs=[
          pl.BlockSpec((BM, fo), lambda i: (i, 0)),
          pl.BlockSpec((BM, fo), lambda i: (i, 0)),
      ],
      out_shape=[
          jax.ShapeDtypeStruct((N, fo), jnp.float32),
          jax.ShapeDtypeStruct((N, fo), jnp.float32),
      ],
  )(x, ws, wn, b)


def _tc_mid_body(sp_ref, a0_ref, a1_ref, d0_ref, d1_ref, ws_ref, wn_ref,
                 b_ref, s_ref, y_ref):
  deg = jnp.maximum(d0_ref[...] + d1_ref[...], 1.0)
  agg = a0_ref[...] + a1_ref[...]
  h = jax.nn.relu(sp_ref[...] + agg / deg[:, None])
  s_ref[...] = jnp.dot(h, ws_ref[...],
                       preferred_element_type=jnp.float32) + b_ref[...]
  y_ref[...] = jnp.dot(h, wn_ref[...], preferred_element_type=jnp.float32)


def _tc_mid(sp, a0, a1, d0, d1, ws, wn, b):
  fi = ws.shape[0]
  fo = ws.shape[1]
  return pl.pallas_call(
      _tc_mid_body,
      grid=(GRID,),
      in_specs=[
          pl.BlockSpec((BM, fi), lambda i: (i, 0)),
          pl.BlockSpec((BM, fi), lambda i: (i, 0)),
          pl.BlockSpec((BM, fi), lambda i: (i, 0)),
          pl.BlockSpec((BM,), lambda i: (i,)),
          pl.BlockSpec((BM,), lambda i: (i,)),
          pl.BlockSpec((fi, fo), lambda i: (0, 0)),
          pl.BlockSpec((fi, fo), lambda i: (0, 0)),
          pl.BlockSpec((1, fo), lambda i: (0, 0)),
      ],
      out_specs=[
          pl.BlockSpec((BM, fo), lambda i: (i, 0)),
          pl.BlockSpec((BM, fo), lambda i: (i, 0)),
      ],
      out_shape=[
          jax.ShapeDtypeStruct((N, fo), jnp.float32),
          jax.ShapeDtypeStruct((N, fo), jnp.float32),
      ],
  )(sp, a0, a1, d0, d1, ws, wn, b)


def _tc_last_body(sp_ref, a0_ref, a1_ref, d0_ref, d1_ref, o_ref):
  deg = jnp.maximum(d0_ref[...] + d1_ref[...], 1.0)
  agg = a0_ref[...] + a1_ref[...]
  o_ref[...] = sp_ref[...] + agg / deg[:, None]


def _tc_last(sp, a0, a1, d0, d1):
  fo = sp.shape[1]
  return pl.pallas_call(
      _tc_last_body,
      grid=(GRID,),
      in_specs=[
          pl.BlockSpec((BM, fo), lambda i: (i, 0)),
          pl.BlockSpec((BM, fo), lambda i: (i, 0)),
          pl.BlockSpec((BM, fo), lambda i: (i, 0)),
          pl.BlockSpec((BM,), lambda i: (i,)),
          pl.BlockSpec((BM,), lambda i: (i,)),
      ],
      out_specs=pl.BlockSpec((BM, fo), lambda i: (i, 0)),
      out_shape=jax.ShapeDtypeStruct((N, fo), jnp.float32),
  )(sp, a0, a1, d0, d1)


def kernel(x, edge_index, W_self0, W_neigh0, b0, W_self1, W_neigh1, b1,
           W_self2, W_neigh2, b2):
  pad = E_PAD - E
  ar = jnp.arange(pad, dtype=jnp.int32)
  pad_pair = jnp.stack([(ar * 97) % N, N + (ar % 8)])
  ei = jnp.concatenate([edge_index, pad_pair], axis=1)
  z2d128 = jnp.zeros((N, F_HID), jnp.float32)
  z1 = jnp.zeros((NPAD,), jnp.float32)
  ws2 = jnp.concatenate([W_self2, jnp.zeros_like(W_self2)], axis=1)
  wn2 = jnp.concatenate([W_neigh2, jnp.zeros_like(W_neigh2)], axis=1)
  b2p = jnp.concatenate([b2, jnp.zeros_like(b2)])

  s0, y0 = _tc_first(x, W_self0, W_neigh0, b0.reshape(1, -1))
  a0, a1, d0, d1 = _make_sc_agg(F_HID, True)(y0, ei, z2d128, z1)
  d0 = d0[:N]
  d1 = d1[:N]
  s1, y1 = _tc_mid(s0, a0, a1, d0, d1, W_self1, W_neigh1, b1.reshape(1, -1))
  a0b, a1b = _make_sc_agg(F_HID, False)(y1, ei, z2d128, z1)
  s2, y2 = _tc_mid(s1, a0b, a1b, d0, d1, ws2, wn2, b2p.reshape(1, -1))
  a0c, a1c = _make_sc_agg(F_HID, False)(y2, ei, z2d128, z1)
  return _tc_last(s2, a0c, a1c, d0, d1)[:, :F_OUT]


# speedup vs baseline: 14.9359x; 1.0356x over previous
import functools

import jax
import jax.numpy as jnp
from jax import lax
from jax.experimental import pallas as pl
from jax.experimental.pallas import tpu as pltpu
from jax.experimental.pallas import tpu_sc as plsc

N = 10000
E = 320000
F_IN = 128
F_HID = 128
F_OUT = 64


NC = 2
NS = 16
NW = NC * NS
C = 128
E_PAD = -(-E // (NW * C)) * NW * C
PT = E_PAD // (NW * C)
NPAD = N + 8
NBUF = 3
DIST = NBUF - 1
UNROLL = 12
R_MAIN = 632
R_LAST = N - (NS - 1) * R_MAIN


@functools.lru_cache(maxsize=None)
def _make_sc_agg(F, compute_deg):
  mesh = plsc.VectorSubcoreMesh(core_axis_name="c", subcore_axis_name="s",
                                num_cores=NC, num_subcores=NS)
  out_type = [jax.ShapeDtypeStruct((N, F), jnp.float32),
              jax.ShapeDtypeStruct((N, F), jnp.float32)]
  nib = 4 if compute_deg else 6
  scratch = (
      [pltpu.VMEM((2, C), jnp.int32) for _ in range(nib)]
      + [pltpu.VMEM((C, F), jnp.float32) for _ in range(NBUF)]
      + [pltpu.VMEM_SHARED((NPAD, F), jnp.float32)]
      + [pltpu.SemaphoreType.DMA for _ in range(nib + 2 * NBUF + 1)]
  )
  if compute_deg:
    out_type += [jax.ShapeDtypeStruct((NPAD,), jnp.float32),
                 jax.ShapeDtypeStruct((NPAD,), jnp.float32)]
    scratch += [
        pltpu.VMEM((C,), jnp.float32),
        pltpu.VMEM_SHARED((NPAD,), jnp.float32),
    ]

  def body(y_hbm, ei_hbm, z2d_hbm, z1_hbm, agg0_hbm, agg1_hbm, *rest):
    if compute_deg:
      deg0_hbm, deg1_hbm = rest[0], rest[1]
      rest = rest[2:]
    eiv = rest[:nib]
    rows = rest[nib:nib + NBUF]
    agg_sh = rest[nib + NBUF]
    sems = rest[nib + NBUF + 1:nib + NBUF + 1 + nib + 2 * NBUF + 1]
    isem = sems[:nib]
    gsem = sems[nib:nib + NBUF]
    ssem = sems[nib + NBUF:nib + 2 * NBUF]
    zsem = sems[nib + 2 * NBUF]
    if compute_deg:
      ones_v, deg_sh = rest[-2], rest[-1]
    c = lax.axis_index("c")
    s = lax.axis_index("s")
    wid = s * NC + c

    r0 = pl.multiple_of(s * R_MAIN, 8)

    def copy_rows(src_ref, dst_ref, sem=None):
      @pl.when(s < NS - 1)
      def _():
        sl = (pl.ds(r0, R_MAIN),)
        if sem is None:
          pltpu.sync_copy(src_ref.at[sl], dst_ref.at[sl])
        else:
          pltpu.async_copy(src_ref.at[sl], dst_ref.at[sl], sem)
      @pl.when(s == NS - 1)
      def _():
        sl = (pl.ds(r0, R_LAST),)
        if sem is None:
          pltpu.sync_copy(src_ref.at[sl], dst_ref.at[sl])
        else:
          pltpu.async_copy(src_ref.at[sl], dst_ref.at[sl], sem)

    def wait_rows(src_ref, dst_ref, sem):
      @pl.when(s < NS - 1)
      def _():
        sl = (pl.ds(r0, R_MAIN),)
        pltpu.make_async_copy(src_ref.at[sl], dst_ref.at[sl], sem).wait()
      @pl.when(s == NS - 1)
      def _():
        sl = (pl.ds(r0, R_LAST),)
        pltpu.make_async_copy(src_ref.at[sl], dst_ref.at[sl], sem).wait()

    def idx_load(j, ib):
      off = pl.multiple_of((j * NW + wid) * C, C)
      pltpu.async_copy(ei_hbm.at[:, pl.ds(off, C)], eiv[ib], isem[ib])

    def idx_wait(j, ib):
      off = pl.multiple_of((j * NW + wid) * C, C)
      pltpu.make_async_copy(ei_hbm.at[:, pl.ds(off, C)], eiv[ib],
                            isem[ib]).wait()

    def drain_scatter(b, ib):
      pltpu.make_async_copy(rows[b], agg_sh.at[eiv[ib].at[1]], ssem[b]).wait()
      if compute_deg:
        pltpu.make_async_copy(ones_v, deg_sh.at[eiv[ib].at[1]], ssem[b]).wait()

    copy_rows(z2d_hbm, agg_sh, zsem)
    for k in range(DIST + 1):
      idx_load(k, k)
    for k in range(DIST):
      idx_wait(k, k)
      pltpu.async_copy(y_hbm.at[eiv[k].at[0]], rows[k], gsem[k])
    wait_rows(z2d_hbm, agg_sh, zsem)
    if compute_deg:
      @pl.when(s == 0)
      def _():
        pltpu.sync_copy(z1_hbm, deg_sh)
      ones16 = jnp.full((16,), 1.0, jnp.float32)
      for j in range(C // 16):
        ones_v[pl.ds(j * 16, 16)] = ones16
    plsc.subcore_barrier()

    def iter_body(j, k):
      b = k % NBUF
      ib = k % nib
      pltpu.make_async_copy(y_hbm.at[eiv[ib].at[0]], rows[b], gsem[b]).wait()
      pltpu.async_copy(rows[b], agg_sh.at[eiv[ib].at[1]], ssem[b], add=True)
      if compute_deg:
        pltpu.async_copy(ones_v, deg_sh.at[eiv[ib].at[1]], ssem[b], add=True)
      @pl.when(j + DIST < PT)
      def _():
        @pl.when(j >= 1)
        def _():
          drain_scatter((k + NBUF - 1) % NBUF, (k + nib - 1) % nib)
        @pl.when(j + DIST + 1 < PT)
        def _():
          idx_load(j + DIST + 1, (k + DIST + 1) % nib)
        idx_wait(j + DIST, (k + DIST) % nib)
        pltpu.async_copy(y_hbm.at[eiv[(k + DIST) % nib].at[0]],
                         rows[(k + DIST) % NBUF], gsem[(k + DIST) % NBUF])

    def block(q, carry):
      for k in range(UNROLL):
        iter_body(q * UNROLL + k, k)
      return carry
    lax.fori_loop(0, PT // UNROLL, block, 0)
    for j in range(UNROLL * (PT // UNROLL), PT):
      iter_body(j, j % UNROLL)
    for j in range(max(0, PT - NBUF), PT):
      drain_scatter(j % NBUF, j % nib)

    plsc.subcore_barrier()

    @pl.when(c == 0)
    def _():
      copy_rows(agg_sh, agg0_hbm)
    @pl.when(c == 1)
    def _():
      copy_rows(agg_sh, agg1_hbm)
    if compute_deg:
      @pl.when((c == 0) & (s == 0))
      def _():
        pltpu.sync_copy(deg_sh, deg0_hbm)
      @pl.when((c == 1) & (s == 0))
      def _():
        pltpu.sync_copy(deg_sh, deg1_hbm)

  return pl.kernel(body, out_type=out_type, mesh=mesh, scratch_types=scratch)


BM = 1024
GRID = pl.cdiv(N, BM)


def _tc_layer_body(relu, h_ref, a0_ref, a1_ref, d0_ref, d1_ref, ws_ref,
                   wn_ref, b_ref, o_ref):
  deg = jnp.maximum(d0_ref[...] + d1_ref[...], 1.0)
  h_neigh = (a0_ref[...] + a1_ref[...]) / deg[:, None]
  o = (jnp.dot(h_ref[...], ws_ref[...], preferred_element_type=jnp.float32)
       + jnp.dot(h_neigh, wn_ref[...], preferred_element_type=jnp.float32)
       + b_ref[...])
  o_ref[...] = jax.nn.relu(o) if relu else o


def _tc_layer(h, a0, a1, d0, d1, ws, wn, b, relu):
  fi = ws.shape[0]
  fo = ws.shape[1]
  return pl.pallas_call(
      functools.partial(_tc_layer_body, relu),
      grid=(GRID,),
      in_specs=[
          pl.BlockSpec((BM, fi), lambda i: (i, 0)),
          pl.BlockSpec((BM, fi), lambda i: (i, 0)),
          pl.BlockSpec((BM, fi), lambda i: (i, 0)),
          pl.BlockSpec((BM,), lambda i: (i,)),
          pl.BlockSpec((BM,), lambda i: (i,)),
          pl.BlockSpec((fi, fo), lambda i: (0, 0)),
          pl.BlockSpec((fi, fo), lambda i: (0, 0)),
          pl.BlockSpec((1, fo), lambda i: (0, 0)),
      ],
      out_specs=pl.BlockSpec((BM, fo), lambda i: (i, 0)),
      out_shape=jax.ShapeDtypeStruct((N, fo), jnp.float32),
  )(h, a0, a1, d0, d1, ws, wn, b)


def kernel(x, edge_index, W_self0, W_neigh0, b0, W_self1, W_neigh1, b1,
           W_self2, W_neigh2, b2):
  pad = E_PAD - E
  ar = jnp.arange(pad, dtype=jnp.int32)
  pad_pair = jnp.stack([(ar * 97) % N, N + (ar % 8)])
  ei = jnp.concatenate([edge_index, pad_pair], axis=1)
  z2d128 = jnp.zeros((N, F_HID), jnp.float32)
  z1 = jnp.zeros((NPAD,), jnp.float32)

  a0, a1, d0, d1 = _make_sc_agg(F_IN, True)(x, ei, z2d128, z1)
  d0 = d0[:N]
  d1 = d1[:N]
  h1 = _tc_layer(x, a0, a1, d0, d1, W_self0, W_neigh0, b0.reshape(1, -1),
                 relu=True)
  a0b, a1b = _make_sc_agg(F_HID, False)(h1, ei, z2d128, z1)
  h2 = _tc_layer(h1, a0b, a1b, d0, d1, W_self1, W_neigh1, b1.reshape(1, -1),
                 relu=True)
  a0c, a1c = _make_sc_agg(F_HID, False)(h2, ei, z2d128, z1)
  return _tc_layer(h2, a0c, a1c, d0, d1, W_self2, W_neigh2, b2.reshape(1, -1),
                   relu=False)


# speedup vs baseline: 15.1859x; 1.0167x over previous
import functools

import jax
import jax.numpy as jnp
from jax import lax
from jax.experimental import pallas as pl
from jax.experimental.pallas import tpu as pltpu
from jax.experimental.pallas import tpu_sc as plsc

N = 10000
E = 320000
F_IN = 128
F_HID = 128
F_OUT = 64


NC = 2
NS = 16
NW = NC * NS
C = 128
E_PAD = -(-E // (NW * C)) * NW * C
PT = E_PAD // (NW * C)
NPAD = N + 8
NBUF = 3
DIST = NBUF - 1
UNROLL = 12
R_MAIN = 632
R_LAST = N - (NS - 1) * R_MAIN


@functools.lru_cache(maxsize=None)
def _make_sc_agg(F, compute_deg):
  mesh = plsc.VectorSubcoreMesh(core_axis_name="c", subcore_axis_name="s",
                                num_cores=NC, num_subcores=NS)
  out_type = [jax.ShapeDtypeStruct((N, F), jnp.float32),
              jax.ShapeDtypeStruct((N, F), jnp.float32)]
  nib = 4 if compute_deg else 6
  scratch = (
      [pltpu.VMEM((2, C), jnp.int32) for _ in range(nib)]
      + [pltpu.VMEM((C, F), jnp.float32) for _ in range(NBUF)]
      + [pltpu.VMEM_SHARED((NPAD, F), jnp.float32)]
      + [pltpu.SemaphoreType.DMA for _ in range(nib + 2 * NBUF + 1)]
  )
  if compute_deg:
    out_type += [jax.ShapeDtypeStruct((NPAD,), jnp.float32),
                 jax.ShapeDtypeStruct((NPAD,), jnp.float32)]
    scratch += [
        pltpu.VMEM((C,), jnp.float32),
        pltpu.VMEM_SHARED((NPAD,), jnp.float32),
    ]

  def body(y_hbm, ei_hbm, et_hbm, z2d_hbm, z1_hbm, agg0_hbm, agg1_hbm, *rest):
    if compute_deg:
      deg0_hbm, deg1_hbm = rest[0], rest[1]
      rest = rest[2:]
    eiv = rest[:nib]
    rows = rest[nib:nib + NBUF]
    agg_sh = rest[nib + NBUF]
    sems = rest[nib + NBUF + 1:nib + NBUF + 1 + nib + 2 * NBUF + 1]
    isem = sems[:nib]
    gsem = sems[nib:nib + NBUF]
    ssem = sems[nib + NBUF:nib + 2 * NBUF]
    zsem = sems[nib + 2 * NBUF]
    if compute_deg:
      ones_v, deg_sh = rest[-2], rest[-1]
    c = lax.axis_index("c")
    s = lax.axis_index("s")
    wid = s * NC + c

    r0 = pl.multiple_of(s * R_MAIN, 8)

    def copy_rows(src_ref, dst_ref, sem=None):
      @pl.when(s < NS - 1)
      def _():
        sl = (pl.ds(r0, R_MAIN),)
        if sem is None:
          pltpu.sync_copy(src_ref.at[sl], dst_ref.at[sl])
        else:
          pltpu.async_copy(src_ref.at[sl], dst_ref.at[sl], sem)
      @pl.when(s == NS - 1)
      def _():
        sl = (pl.ds(r0, R_LAST),)
        if sem is None:
          pltpu.sync_copy(src_ref.at[sl], dst_ref.at[sl])
        else:
          pltpu.async_copy(src_ref.at[sl], dst_ref.at[sl], sem)

    def wait_rows(src_ref, dst_ref, sem):
      @pl.when(s < NS - 1)
      def _():
        sl = (pl.ds(r0, R_MAIN),)
        pltpu.make_async_copy(src_ref.at[sl], dst_ref.at[sl], sem).wait()
      @pl.when(s == NS - 1)
      def _():
        sl = (pl.ds(r0, R_LAST),)
        pltpu.make_async_copy(src_ref.at[sl], dst_ref.at[sl], sem).wait()

    def idx_src(j, tail):
      if tail:
        return et_hbm.at[:, pl.ds(pl.multiple_of(wid * C, C), C)]
      off = pl.multiple_of((j * NW + wid) * C, C)
      return ei_hbm.at[:, pl.ds(off, C)]

    def idx_load(j, ib, tail=False):
      pltpu.async_copy(idx_src(j, tail), eiv[ib], isem[ib])

    def idx_wait(j, ib, tail=False):
      pltpu.make_async_copy(idx_src(j, tail), eiv[ib], isem[ib]).wait()

    def drain_scatter(b, ib):
      pltpu.make_async_copy(rows[b], agg_sh.at[eiv[ib].at[1]], ssem[b]).wait()
      if compute_deg:
        pltpu.make_async_copy(ones_v, deg_sh.at[eiv[ib].at[1]], ssem[b]).wait()

    copy_rows(z2d_hbm, agg_sh, zsem)
    for k in range(DIST + 1):
      idx_load(k, k)
    for k in range(DIST):
      idx_wait(k, k)
      pltpu.async_copy(y_hbm.at[eiv[k].at[0]], rows[k], gsem[k])
    wait_rows(z2d_hbm, agg_sh, zsem)
    if compute_deg:
      @pl.when(s == 0)
      def _():
        pltpu.sync_copy(z1_hbm, deg_sh)
      ones16 = jnp.full((16,), 1.0, jnp.float32)
      for j in range(C // 16):
        ones_v[pl.ds(j * 16, 16)] = ones16
    plsc.subcore_barrier()

    def iter_body(j, k, tail_load=False, tail_wait=False):
      b = k % NBUF
      ib = k % nib
      pltpu.make_async_copy(y_hbm.at[eiv[ib].at[0]], rows[b], gsem[b]).wait()
      pltpu.async_copy(rows[b], agg_sh.at[eiv[ib].at[1]], ssem[b], add=True)
      if compute_deg:
        pltpu.async_copy(ones_v, deg_sh.at[eiv[ib].at[1]], ssem[b], add=True)
      @pl.when(j + DIST < PT)
      def _():
        @pl.when(j >= 1)
        def _():
          drain_scatter((k + NBUF - 1) % NBUF, (k + nib - 1) % nib)
        @pl.when(j + DIST + 1 < PT)
        def _():
          idx_load(j + DIST + 1, (k + DIST + 1) % nib, tail_load)
        idx_wait(j + DIST, (k + DIST) % nib, tail_wait)
        pltpu.async_copy(y_hbm.at[eiv[(k + DIST) % nib].at[0]],
                         rows[(k + DIST) % NBUF], gsem[(k + DIST) % NBUF])

    def block(q, carry):
      for k in range(UNROLL):
        iter_body(q * UNROLL + k, k)
      return carry
    lax.fori_loop(0, PT // UNROLL, block, 0)
    for j in range(UNROLL * (PT // UNROLL), PT):
      iter_body(j, j % UNROLL, tail_load=(j + DIST + 1 == PT - 1),
                tail_wait=(j + DIST == PT - 1))
    for j in range(max(0, PT - NBUF), PT):
      drain_scatter(j % NBUF, j % nib)

    plsc.subcore_barrier()

    @pl.when(c == 0)
    def _():
      copy_rows(agg_sh, agg0_hbm)
    @pl.when(c == 1)
    def _():
      copy_rows(agg_sh, agg1_hbm)
    if compute_deg:
      @pl.when((c == 0) & (s == 0))
      def _():
        pltpu.sync_copy(deg_sh, deg0_hbm)
      @pl.when((c == 1) & (s == 0))
      def _():
        pltpu.sync_copy(deg_sh, deg1_hbm)

  return pl.kernel(body, out_type=out_type, mesh=mesh, scratch_types=scratch)


BM = 1024
GRID = pl.cdiv(N, BM)


def _tc_layer_body(relu, h_ref, a0_ref, a1_ref, d0_ref, d1_ref, ws_ref,
                   wn_ref, b_ref, o_ref):
  deg = jnp.maximum(d0_ref[...] + d1_ref[...], 1.0)
  h_neigh = (a0_ref[...] + a1_ref[...]) / deg[:, None]
  o = (jnp.dot(h_ref[...], ws_ref[...], preferred_element_type=jnp.float32)
       + jnp.dot(h_neigh, wn_ref[...], preferred_element_type=jnp.float32)
       + b_ref[...])
  o_ref[...] = jax.nn.relu(o) if relu else o


def _tc_layer(h, a0, a1, d0, d1, ws, wn, b, relu):
  fi = ws.shape[0]
  fo = ws.shape[1]
  return pl.pallas_call(
      functools.partial(_tc_layer_body, relu),
      grid=(GRID,),
      in_specs=[
          pl.BlockSpec((BM, fi), lambda i: (i, 0)),
          pl.BlockSpec((BM, fi), lambda i: (i, 0)),
          pl.BlockSpec((BM, fi), lambda i: (i, 0)),
          pl.BlockSpec((BM,), lambda i: (i,)),
          pl.BlockSpec((BM,), lambda i: (i,)),
          pl.BlockSpec((fi, fo), lambda i: (0, 0)),
          pl.BlockSpec((fi, fo), lambda i: (0, 0)),
          pl.BlockSpec((1, fo), lambda i: (0, 0)),
      ],
      out_specs=pl.BlockSpec((BM, fo), lambda i: (i, 0)),
      out_shape=jax.ShapeDtypeStruct((N, fo), jnp.float32),
  )(h, a0, a1, d0, d1, ws, wn, b)


def kernel(x, edge_index, W_self0, W_neigh0, b0, W_self1, W_neigh1, b1,
           W_self2, W_neigh2, b2):
  pad = E_PAD - E
  tail_off = (PT - 1) * NW * C
  ar = jnp.arange(pad, dtype=jnp.int32)
  pad_pair = jnp.stack([(ar * 97) % N, N + (ar % 8)])
  ei_tail = jnp.concatenate([edge_index[:, tail_off:], pad_pair], axis=1)
  z2d128 = jnp.zeros((N, F_HID), jnp.float32)
  z1 = jnp.zeros((NPAD,), jnp.float32)

  a0, a1, d0, d1 = _make_sc_agg(F_IN, True)(x, edge_index, ei_tail, z2d128, z1)
  d0 = d0[:N]
  d1 = d1[:N]
  h1 = _tc_layer(x, a0, a1, d0, d1, W_self0, W_neigh0, b0.reshape(1, -1),
                 relu=True)
  a0b, a1b = _make_sc_agg(F_HID, False)(h1, edge_index, ei_tail, z2d128, z1)
  h2 = _tc_layer(h1, a0b, a1b, d0, d1, W_self1, W_neigh1, b1.reshape(1, -1),
                 relu=True)
  a0c, a1c = _make_sc_agg(F_HID, False)(h2, edge_index, ei_tail, z2d128, z1)
  return _tc_layer(h2, a0c, a1c, d0, d1, W_self2, W_neigh2, b2.reshape(1, -1),
                   relu=False)
